# baseline (speedup 1.0000x reference)
import functools
import math

import jax
import jax.numpy as jnp
from jax import lax
from jax.experimental import pallas as pl
from jax.experimental.pallas import tpu as pltpu
from jax.experimental.pallas import tpu_sc as plsc

V = 32000
N = 4096
_SMOOTH = 0.1 / (V - 2)
_CONF = 0.9
_C_ENT = (V - 2) * _SMOOTH * math.log(_SMOOTH) + _CONF * math.log(_CONF)

_BC = 1280
_NC, _NS = 2, 16
_NW = _NC * _NS
_BPW = N // _NW
_L = 16


def _rowsum_body(x_ref, t_ref, out_ref):
    j = pl.program_id(0)
    x = x_ref[...]
    t = t_ref[...]
    validf = jnp.where(t != 0, 1.0, 0.0)
    rs = jnp.sum(x, axis=1, keepdims=True)
    partial = -_SMOOTH * jnp.sum(validf * rs, keepdims=True)

    @pl.when(j == 0)
    def _init():
        out_ref[...] = (_C_ENT * jnp.sum(validf, keepdims=True)
                        + _SMOOTH * jnp.sum(validf * x[:, 0:1], keepdims=True))

    out_ref[...] += partial


def _gather_body(flat_hbm, tgt_hbm, out_hbm, tgt_v, idx_v, val_v, acc_v, sem):
    wid = lax.axis_index("s") * _NC + lax.axis_index("c")
    base = wid * _BPW
    pltpu.sync_copy(tgt_hbm.at[pl.ds(base, _BPW)], tgt_v)
    for k in range(_BPW // _L):
        t16 = tgt_v[pl.ds(k * _L, _L)]
        rows = lax.iota(jnp.int32, _L) + (base + k * _L)
        idx_v[pl.ds(k * _L, _L)] = rows * V + t16
    pltpu.async_copy(flat_hbm.at[idx_v], val_v, sem).wait()
    acc = jnp.zeros((_L,), jnp.float32)
    for k in range(_BPW // _L):
        t16 = tgt_v[pl.ds(k * _L, _L)]
        v16 = val_v[pl.ds(k * _L, _L)]
        acc = acc + jnp.where(t16 == 0, 0.0, v16)
    acc_v[...] = acc
    pltpu.sync_copy(acc_v, out_hbm.at[wid])


_gather_kernel = functools.partial(
    pl.kernel,
    out_type=jax.ShapeDtypeStruct((_NW, _L), jnp.float32),
    mesh=plsc.VectorSubcoreMesh(
        core_axis_name="c", subcore_axis_name="s",
        num_cores=_NC, num_subcores=_NS),
    scratch_types=[
        pltpu.VMEM((_BPW,), jnp.int32),
        pltpu.VMEM((_BPW,), jnp.int32),
        pltpu.VMEM((_BPW,), jnp.float32),
        pltpu.VMEM((_L,), jnp.float32),
        pltpu.SemaphoreType.DMA,
    ],
)(_gather_body)


def kernel(output, target):
    t32 = target.astype(jnp.int32)
    t2 = t32.reshape(N, 1)
    flat = output.reshape(N * V)
    sweep = pl.pallas_call(
        _rowsum_body,
        grid=(V // _BC,),
        in_specs=[
            pl.BlockSpec((N, _BC), lambda j: (0, j)),
            pl.BlockSpec((N, 1), lambda j: (0, 0)),
        ],
        out_specs=pl.BlockSpec((1, 1), lambda j: (0, 0)),
        out_shape=jax.ShapeDtypeStruct((1, 1), jnp.float32),
    )(output, t2)
    partials = _gather_kernel(flat, t32)
    return sweep[0, 0] - (_CONF - _SMOOTH) * jnp.sum(partials)


# speedup vs baseline: 3.3074x; 3.3074x over previous
import math

import jax
import jax.numpy as jnp
from jax.experimental import pallas as pl

V = 32000
N = 4096
_SMOOTH = 0.1 / (V - 2)
_CONF = 0.9
_C_ENT = (V - 2) * _SMOOTH * math.log(_SMOOTH) + _CONF * math.log(_CONF)

_BC = 1280


def _loss_body(x_ref, t_ref, out_ref):
    j = pl.program_id(0)
    x = x_ref[...]
    t = t_ref[...]
    validf = jnp.where(t != 0, 1.0, 0.0)
    col = jax.lax.broadcasted_iota(jnp.int32, (N, _BC), 1)
    rs = jnp.sum(x, axis=1, keepdims=True)
    mt = jnp.sum(jnp.where(col == t - j * _BC, x, 0.0),
                 axis=1, keepdims=True)
    partial = -jnp.sum(validf * (_SMOOTH * rs + (_CONF - _SMOOTH) * mt),
                       keepdims=True)

    @pl.when(j == 0)
    def _init():
        out_ref[...] = (_C_ENT * jnp.sum(validf, keepdims=True)
                        + _SMOOTH * jnp.sum(validf * x[:, 0:1], keepdims=True))

    out_ref[...] += partial


def kernel(output, target):
    t2 = target.astype(jnp.int32).reshape(N, 1)
    res = pl.pallas_call(
        _loss_body,
        grid=(V // _BC,),
        in_specs=[
            pl.BlockSpec((N, _BC), lambda j: (0, j)),
            pl.BlockSpec((N, 1), lambda j: (0, 0)),
        ],
        out_specs=pl.BlockSpec((1, 1), lambda j: (0, 0)),
        out_shape=jax.ShapeDtypeStruct((1, 1), jnp.float32),
    )(output, t2)
    return res[0, 0]
